# baseline (speedup 1.0000x reference)
import functools

import jax
import jax.numpy as jnp
from jax import lax
from jax.experimental import pallas as pl
from jax.experimental.pallas import tpu as pltpu
from jax.experimental.pallas import tpu_sc as plsc

EMBED_DIM = 32
LANES = 128
PACK = LANES // EMBED_DIM
NUM_CORES = 2
NUM_SUBCORES = 16
NUM_WORKERS = NUM_CORES * NUM_SUBCORES
NBUF = 4
CHUNK = 64
OUT_ROWS_PER_CHUNK = CHUNK * EMBED_DIM // LANES


def _make_gather(n_idx: int):
  idx_rows = n_idx // LANES
  rows_per_w = idx_rows // NUM_WORKERS
  chunks_per_w = rows_per_w * LANES // CHUNK
  n_groups = chunks_per_w // NBUF
  out_rows = n_idx * EMBED_DIM // LANES
  mesh = plsc.VectorSubcoreMesh(core_axis_name="c", subcore_axis_name="s")

  @functools.partial(
      pl.kernel,
      mesh=mesh,
      compiler_params=pltpu.CompilerParams(
          use_tc_tiling_on_sc=True, needs_layout_passes=False),
      out_type=jax.ShapeDtypeStruct((out_rows, LANES), jnp.float32),
      scratch_types=(
          [pltpu.VMEM((rows_per_w, LANES), jnp.int32)]
          + [pltpu.VMEM((CHUNK,), jnp.int32) for _ in range(NBUF)]
          + [pltpu.VMEM((CHUNK, LANES), jnp.float32) for _ in range(NBUF)]
          + [pltpu.VMEM((OUT_ROWS_PER_CHUNK, LANES), jnp.float32)
             for _ in range(NBUF)]
          + [pltpu.SemaphoreType.DMA for _ in range(2 * NBUF)]
      ),
  )
  def k(table_hbm, idx_hbm, out_hbm, idx_v, *scratch):
    qv = scratch[:NBUF]
    bufa = scratch[NBUF:2 * NBUF]
    bufb = scratch[2 * NBUF:3 * NBUF]
    gsem = scratch[3 * NBUF:4 * NBUF]
    ssem = scratch[4 * NBUF:]
    wid = lax.axis_index("s") * NUM_CORES + lax.axis_index("c")
    chunk_base = wid * chunks_per_w
    halves = LANES // CHUNK

    pltpu.sync_copy(idx_hbm.at[pl.ds(wid * rows_per_w, rows_per_w), :], idx_v)

    def idx_vec(c, q):
      return idx_v[c // halves, pl.ds((c % halves) * CHUNK + 16 * q, 16)]

    def fill_qidx(c, b):
      for q in range(CHUNK // 16):
        qv[b][pl.ds(16 * q, 16)] = lax.shift_right_logical(idx_vec(c, q), 2)

    def start_gather(b):
      pltpu.async_copy(table_hbm.at[qv[b]], bufa[b], gsem[b])

    def wait_gather(b):
      pltpu.make_async_copy(table_hbm.at[qv[b]], bufa[b], gsem[b]).wait()

    def extract(c, b):
      for q in range(CHUNK // 16):
        jv = 16 * q + lax.iota(jnp.int32, 16)
        sv = lax.shift_left(
            jnp.bitwise_and(idx_vec(c, q), jnp.int32(PACK - 1)), 5)
        for c32 in range(EMBED_DIM):
          colv = sv + c32
          val = plsc.load_gather(bufa[b], [jv, colv])
          fv = 32 * jv + c32
          plsc.store_scatter(
              bufb[b],
              [lax.shift_right_logical(fv, 7), jnp.bitwise_and(fv, 127)],
              val)

    def start_store(c, b):
      pltpu.async_copy(
          bufb[b],
          out_hbm.at[pl.ds((chunk_base + c) * OUT_ROWS_PER_CHUNK,
                           OUT_ROWS_PER_CHUNK), :],
          ssem[b])

    def wait_store(b):
      pltpu.make_async_copy(
          bufb[b], out_hbm.at[pl.ds(0, OUT_ROWS_PER_CHUNK), :],
          ssem[b]).wait()

    for b in range(NBUF):
      fill_qidx(b, b)
      start_gather(b)

    def body(j, carry):
      c0 = j * NBUF
      for b in range(NBUF):
        wait_gather(b)
        extract(c0 + b, b)
        start_store(c0 + b, b)
      for b in range(NBUF):
        wait_store(b)
        fill_qidx(c0 + NBUF + b, b)
        start_gather(b)
      return carry

    lax.fori_loop(0, n_groups - 1, body, 0)

    c0 = (n_groups - 1) * NBUF
    for b in range(NBUF):
      wait_gather(b)
      extract(c0 + b, b)
      start_store(c0 + b, b)
    for b in range(NBUF):
      wait_store(b)

  return k


def kernel(indices, table):
  b, h = indices.shape
  n, d = table.shape
  idx128 = indices.reshape(b * h // LANES, LANES)
  table128 = table.reshape(n // PACK, LANES)
  gather = _make_gather(b * h)
  out = gather(table128, idx128)
  return out.reshape(b, h, EMBED_DIM)


# speedup vs baseline: 2.0151x; 2.0151x over previous
import functools

import jax
import jax.numpy as jnp
from jax import lax
from jax.experimental import pallas as pl
from jax.experimental.pallas import tpu as pltpu
from jax.experimental.pallas import tpu_sc as plsc

EMBED_DIM = 32
NUM_CORES = 2
NUM_SUBCORES = 16
NUM_WORKERS = NUM_CORES * NUM_SUBCORES
NBUF = 8


def _make_gather(batch: int, hist: int):
  rows_per_w = batch // NUM_WORKERS
  n_groups = rows_per_w // NBUF
  mesh = plsc.VectorSubcoreMesh(core_axis_name="c", subcore_axis_name="s")

  @functools.partial(
      pl.kernel,
      mesh=mesh,
      compiler_params=pltpu.CompilerParams(use_tc_tiling_on_sc=False),
      out_type=jax.ShapeDtypeStruct((batch, hist, EMBED_DIM), jnp.float32),
      scratch_types=(
          [pltpu.VMEM((rows_per_w, hist), jnp.int32)]
          + [pltpu.VMEM((hist, EMBED_DIM), jnp.float32) for _ in range(NBUF)]
          + [pltpu.SemaphoreType.DMA for _ in range(2 * NBUF)]
      ),
  )
  def k(table_hbm, idx_hbm, out_hbm, idx_v, *bufs_and_sems):
    bufs = bufs_and_sems[:NBUF]
    gsem = bufs_and_sems[NBUF:2 * NBUF]
    ssem = bufs_and_sems[2 * NBUF:]
    wid = lax.axis_index("s") * NUM_CORES + lax.axis_index("c")
    row_base = wid * rows_per_w

    pltpu.sync_copy(idx_hbm.at[pl.ds(row_base, rows_per_w), :], idx_v)

    def start_gather(c, b):
      pltpu.async_copy(table_hbm.at[idx_v.at[c]], bufs[b], gsem[b])

    def wait_gather(b):
      pltpu.make_async_copy(
          table_hbm.at[idx_v.at[0]], bufs[b], gsem[b]).wait()

    def start_store(c, b):
      pltpu.async_copy(bufs[b], out_hbm.at[row_base + c], ssem[b])

    def wait_store(b):
      pltpu.make_async_copy(bufs[b], out_hbm.at[row_base], ssem[b]).wait()

    for b in range(NBUF):
      start_gather(b, b)

    def body(j, carry):
      c0 = j * NBUF
      for b in range(NBUF):
        wait_gather(b)
        start_store(c0 + b, b)
      for b in range(NBUF):
        wait_store(b)
        start_gather(c0 + NBUF + b, b)
      return carry

    lax.fori_loop(0, n_groups - 1, body, 0)

    c0 = (n_groups - 1) * NBUF
    for b in range(NBUF):
      wait_gather(b)
      start_store(c0 + b, b)
    for b in range(NBUF):
      wait_store(b)

  return k


def kernel(indices, table):
  b, h = indices.shape
  gather = _make_gather(b, h)
  return gather(table, indices)
